# baseline (speedup 1.0000x reference)
import functools

import jax
import jax.numpy as jnp
from jax import lax
from jax.experimental import pallas as pl
from jax.experimental.pallas import tpu as pltpu
from jax.experimental.pallas import tpu_sc as plsc

N_NODES = 10000
D_IN = 256
D_HID = 512
N_SC = 2
N_TILES = 16
CHUNK = 128
N_CHUNKS = 79
EDGES_PER_TILE = CHUNK * N_CHUNKS
E_PAD = EDGES_PER_TILE * N_TILES
ACC_ROWS = 10240
ROWS_PER_TILE = ACC_ROWS // N_TILES
HALF = D_IN // 2


def _sc_aggregate(x2, src3, dst3):
    @functools.partial(
        pl.kernel,
        mesh=plsc.VectorSubcoreMesh(core_axis_name="c", subcore_axis_name="s"),
        out_type=jax.ShapeDtypeStruct((N_SC, ACC_ROWS, HALF), jnp.float32),
        scratch_types=[
            pltpu.VMEM((40, CHUNK), jnp.int32),
            pltpu.VMEM((40, CHUNK), jnp.int32),
            pltpu.VMEM((CHUNK, HALF), jnp.float32),
            pltpu.VMEM((CHUNK, HALF), jnp.float32),
            pltpu.VMEM_SHARED((ACC_ROWS, HALF), jnp.float32),
            pltpu.SemaphoreType.DMA,
            pltpu.SemaphoreType.DMA,
            pltpu.SemaphoreType.DMA,
            pltpu.SemaphoreType.DMA,
        ],
    )
    def k(x2_hbm, src_hbm, dst_hbm, out_hbm, src_v, dst_v, rows0, rows1,
          acc_sh, sem0, sem1, ssem0, ssem1):
        c = lax.axis_index("c")
        s = lax.axis_index("s")

        def _zrow(i, _):
            def _zlane(l, _):
                rows0[i, pl.ds(l * 16, 16)] = jnp.zeros((16,), jnp.float32)
                return 0
            return lax.fori_loop(0, HALF // 16, _zlane, 0)

        lax.fori_loop(0, CHUNK, _zrow, 0)
        for kk in range(ROWS_PER_TILE // CHUNK):
            pltpu.sync_copy(
                rows0, acc_sh.at[pl.ds(s * ROWS_PER_TILE + kk * CHUNK, CHUNK)])
        plsc.subcore_barrier()

        for ph, nch in ((0, 40), (1, 39)):
            pltpu.sync_copy(src_hbm.at[c, s, pl.ds(ph * 40, nch)],
                            src_v.at[pl.ds(0, nch)])
            pltpu.sync_copy(dst_hbm.at[s, pl.ds(ph * 40, nch)],
                            dst_v.at[pl.ds(0, nch)])
            pltpu.async_copy(x2_hbm.at[src_v.at[0]], rows0, sem0)
            pltpu.async_copy(x2_hbm.at[src_v.at[1]], rows1, sem1)
            npairs = nch // 2

            def _step(jj, _):
                j0 = 2 * jj
                j1 = j0 + 1
                pltpu.make_async_copy(
                    x2_hbm.at[src_v.at[j0]], rows0, sem0).wait()
                pltpu.async_copy(rows0, acc_sh.at[dst_v.at[j0]], ssem0,
                                 add=True)
                pltpu.make_async_copy(
                    x2_hbm.at[src_v.at[j1]], rows1, sem1).wait()
                pltpu.async_copy(rows1, acc_sh.at[dst_v.at[j1]], ssem1,
                                 add=True)

                @pl.when(j0 + 2 < nch)
                def _():
                    pltpu.make_async_copy(
                        rows0, acc_sh.at[dst_v.at[j0]], ssem0).wait()
                    pltpu.async_copy(x2_hbm.at[src_v.at[j0 + 2]], rows0, sem0)

                @pl.when(j1 + 2 < nch)
                def _():
                    pltpu.make_async_copy(
                        rows1, acc_sh.at[dst_v.at[j1]], ssem1).wait()
                    pltpu.async_copy(x2_hbm.at[src_v.at[j1 + 2]], rows1, sem1)
                return 0

            lax.fori_loop(0, npairs, _step, 0)
            if nch % 2 == 1:
                j_last = nch - 1
                pltpu.make_async_copy(
                    x2_hbm.at[src_v.at[j_last]], rows0, sem0).wait()
                pltpu.async_copy(rows0, acc_sh.at[dst_v.at[j_last]], ssem0,
                                 add=True)
            jl0 = nch - 1 if nch % 2 == 1 else nch - 2
            jl1 = nch - 2 if nch % 2 == 1 else nch - 1
            pltpu.make_async_copy(rows0, acc_sh.at[dst_v.at[jl0]], ssem0).wait()
            pltpu.make_async_copy(rows1, acc_sh.at[dst_v.at[jl1]], ssem1).wait()
        plsc.subcore_barrier()
        for kk in range(ROWS_PER_TILE // CHUNK):
            off = s * ROWS_PER_TILE + kk * CHUNK
            pltpu.sync_copy(acc_sh.at[pl.ds(off, CHUNK)],
                            out_hbm.at[c, pl.ds(off, CHUNK)])

    return k(x2, src3, dst3)


_HIGH = jax.lax.Precision.DEFAULT


def _tc_layer1(epsv, x, aggr, W1, b1):
    blk = 2000

    def body(eps_ref, x_ref, aL_ref, aR_ref, w_ref, b_ref, y_ref, s_ref, q_ref):
        i = pl.program_id(0)
        e = eps_ref[0, 0]
        h = (1.0 + e) * x_ref[...] + jnp.concatenate(
            [aL_ref[0], aR_ref[0]], axis=1)
        y = jnp.dot(h, w_ref[...], preferred_element_type=jnp.float32,
                    precision=_HIGH) + b_ref[...]
        y_ref[...] = y

        @pl.when(i == 0)
        def _():
            s_ref[...] = jnp.zeros_like(s_ref)
            q_ref[...] = jnp.zeros_like(q_ref)

        s_ref[...] += jnp.sum(y, axis=0, keepdims=True)
        q_ref[...] += jnp.sum(y * y, axis=0, keepdims=True)

    return pl.pallas_call(
        body,
        grid=(N_NODES // blk,),
        in_specs=[
            pl.BlockSpec((1, 1), lambda i: (0, 0), memory_space=pltpu.SMEM),
            pl.BlockSpec((blk, D_IN), lambda i: (i, 0)),
            pl.BlockSpec((1, blk, HALF), lambda i: (0, i, 0)),
            pl.BlockSpec((1, blk, HALF), lambda i: (1, i, 0)),
            pl.BlockSpec((D_IN, D_HID), lambda i: (0, 0)),
            pl.BlockSpec((1, D_HID), lambda i: (0, 0)),
        ],
        out_specs=[
            pl.BlockSpec((blk, D_HID), lambda i: (i, 0)),
            pl.BlockSpec((1, D_HID), lambda i: (0, 0)),
            pl.BlockSpec((1, D_HID), lambda i: (0, 0)),
        ],
        out_shape=[
            jax.ShapeDtypeStruct((N_NODES, D_HID), jnp.float32),
            jax.ShapeDtypeStruct((1, D_HID), jnp.float32),
            jax.ShapeDtypeStruct((1, D_HID), jnp.float32),
        ],
    )(epsv, x, aggr, aggr, W1, b1)


def _bn_affine(s, q, g, be):
    m = s * (1.0 / N_NODES)
    v = q * (1.0 / N_NODES) - m * m
    a = g * jax.lax.rsqrt(v + 1e-5)
    return a, be - m * a


def _tc_layer2(y1, s1, q1, g1, be1, W2, b2):
    blk = 2000

    def body(y_ref, s1_ref, q1_ref, g_ref, be_ref, w_ref, b_ref,
             y2_ref, s_ref, q_ref):
        i = pl.program_id(0)
        a, b0 = _bn_affine(s1_ref[...], q1_ref[...], g_ref[...], be_ref[...])
        z = jnp.maximum(y_ref[...] * a + b0, 0.0)
        y2 = jnp.dot(z, w_ref[...], preferred_element_type=jnp.float32,
                     precision=_HIGH) + b_ref[...]
        y2_ref[...] = y2

        @pl.when(i == 0)
        def _():
            s_ref[...] = jnp.zeros_like(s_ref)
            q_ref[...] = jnp.zeros_like(q_ref)

        s_ref[...] += jnp.sum(y2, axis=0, keepdims=True)
        q_ref[...] += jnp.sum(y2 * y2, axis=0, keepdims=True)

    vec = pl.BlockSpec((1, D_HID), lambda i: (0, 0))
    return pl.pallas_call(
        body,
        grid=(N_NODES // blk,),
        in_specs=[
            pl.BlockSpec((blk, D_HID), lambda i: (i, 0)),
            vec, vec, vec, vec,
            pl.BlockSpec((D_HID, D_HID), lambda i: (0, 0)),
            vec,
        ],
        out_specs=[
            pl.BlockSpec((blk, D_HID), lambda i: (i, 0)),
            vec, vec,
        ],
        out_shape=[
            jax.ShapeDtypeStruct((N_NODES, D_HID), jnp.float32),
            jax.ShapeDtypeStruct((1, D_HID), jnp.float32),
            jax.ShapeDtypeStruct((1, D_HID), jnp.float32),
        ],
    )(y1, s1, q1, g1, be1, W2, b2)


def _tc_layer3(y2, s2, q2, g2, be2):
    blk = 2000

    def body(y_ref, s2_ref, q2_ref, g_ref, be_ref, o_ref):
        a, b0 = _bn_affine(s2_ref[...], q2_ref[...], g_ref[...], be_ref[...])
        o_ref[...] = jnp.maximum(y_ref[...] * a + b0, 0.0)

    vec = pl.BlockSpec((1, D_HID), lambda i: (0, 0))
    return pl.pallas_call(
        body,
        grid=(N_NODES // blk,),
        in_specs=[
            pl.BlockSpec((blk, D_HID), lambda i: (i, 0)),
            vec, vec, vec, vec,
        ],
        out_specs=pl.BlockSpec((blk, D_HID), lambda i: (i, 0)),
        out_shape=jax.ShapeDtypeStruct((N_NODES, D_HID), jnp.float32),
    )(y2, s2, q2, g2, be2)


def kernel(x, edge_index, eps, W1, b1, g1, be1, W2, b2, g2, be2):
    E = edge_index.shape[1]
    src = edge_index[0]
    dst = edge_index[1]

    pad = E_PAD - E
    src_p = jnp.concatenate([2 * src, jnp.zeros((pad,), jnp.int32)])
    dst_p = jnp.concatenate([dst, jnp.full((pad,), N_NODES, jnp.int32)])
    src3 = jnp.stack([src_p, src_p + 1]).reshape(
        N_SC, N_TILES, N_CHUNKS, CHUNK)
    dst3 = dst_p.reshape(N_TILES, N_CHUNKS, CHUNK)
    x2 = x.reshape(2 * N_NODES, HALF)

    aggr = _sc_aggregate(x2, src3, dst3)

    epsv = jnp.reshape(eps, (1, 1))
    y1, s1, q1 = _tc_layer1(epsv, x, aggr, W1, jnp.reshape(b1, (1, D_HID)))
    g1v = jnp.reshape(g1, (1, D_HID))
    be1v = jnp.reshape(be1, (1, D_HID))
    y2, s2, q2 = _tc_layer2(y1, s1, q1, g1v, be1v, W2,
                            jnp.reshape(b2, (1, D_HID)))
    g2v = jnp.reshape(g2, (1, D_HID))
    be2v = jnp.reshape(be2, (1, D_HID))
    return _tc_layer3(y2, s2, q2, g2v, be2v)


# speedup vs baseline: 1.1789x; 1.1789x over previous
import functools

import jax
import jax.numpy as jnp
from jax import lax
from jax.experimental import pallas as pl
from jax.experimental.pallas import tpu as pltpu
from jax.experimental.pallas import tpu_sc as plsc

N_NODES = 10000
D_IN = 256
D_HID = 512
N_SC = 2
N_TILES = 16
CHUNK = 128
N_CHUNKS = 79
EDGES_PER_TILE = CHUNK * N_CHUNKS
E_PAD = EDGES_PER_TILE * N_TILES
ACC_ROWS = 10240
ROWS_PER_TILE = ACC_ROWS // N_TILES
HALF = D_IN // 2


def _sc_aggregate(x2, src3, dst3):
    @functools.partial(
        pl.kernel,
        mesh=plsc.VectorSubcoreMesh(core_axis_name="c", subcore_axis_name="s"),
        out_type=jax.ShapeDtypeStruct((N_SC, ACC_ROWS, HALF), jnp.float32),
        scratch_types=[
            pltpu.VMEM((40, CHUNK), jnp.int32),
            pltpu.VMEM((40, CHUNK), jnp.int32),
            pltpu.VMEM((CHUNK, HALF), jnp.float32),
            pltpu.VMEM((CHUNK, HALF), jnp.float32),
            pltpu.VMEM_SHARED((ACC_ROWS, HALF), jnp.float32),
            pltpu.SemaphoreType.DMA,
            pltpu.SemaphoreType.DMA,
        ],
    )
    def k(x2_hbm, src_hbm, dst_hbm, out_hbm, src_v, dst_v, rows0, rows1,
          acc_sh, sem0, sem1):
        c = lax.axis_index("c")
        s = lax.axis_index("s")

        def _zrow(i, _):
            def _zlane(l, _):
                rows0[i, pl.ds(l * 16, 16)] = jnp.zeros((16,), jnp.float32)
                return 0
            return lax.fori_loop(0, HALF // 16, _zlane, 0)

        lax.fori_loop(0, CHUNK, _zrow, 0)
        for kk in range(ROWS_PER_TILE // CHUNK):
            pltpu.sync_copy(
                rows0, acc_sh.at[pl.ds(s * ROWS_PER_TILE + kk * CHUNK, CHUNK)])
        plsc.subcore_barrier()

        for ph, nch in ((0, 40), (1, 39)):
            pltpu.sync_copy(src_hbm.at[c, s, pl.ds(ph * 40, nch)],
                            src_v.at[pl.ds(0, nch)])
            pltpu.sync_copy(dst_hbm.at[s, pl.ds(ph * 40, nch)],
                            dst_v.at[pl.ds(0, nch)])
            pltpu.async_copy(x2_hbm.at[src_v.at[0]], rows0, sem0)
            npairs = nch // 2

            def _step(jj, _):
                j0 = 2 * jj
                j1 = j0 + 1
                pltpu.async_copy(x2_hbm.at[src_v.at[j1]], rows1, sem1)
                pltpu.make_async_copy(
                    x2_hbm.at[src_v.at[j0]], rows0, sem0).wait()
                pltpu.sync_copy(rows0, acc_sh.at[dst_v.at[j0]], add=True)

                @pl.when(j0 + 2 < nch)
                def _():
                    pltpu.async_copy(x2_hbm.at[src_v.at[j0 + 2]], rows0, sem0)

                pltpu.make_async_copy(
                    x2_hbm.at[src_v.at[j1]], rows1, sem1).wait()
                pltpu.sync_copy(rows1, acc_sh.at[dst_v.at[j1]], add=True)
                return 0

            lax.fori_loop(0, npairs, _step, 0)
            if nch % 2 == 1:
                j_last = nch - 1
                pltpu.make_async_copy(
                    x2_hbm.at[src_v.at[j_last]], rows0, sem0).wait()
                pltpu.sync_copy(rows0, acc_sh.at[dst_v.at[j_last]], add=True)
        plsc.subcore_barrier()
        for kk in range(ROWS_PER_TILE // CHUNK):
            off = s * ROWS_PER_TILE + kk * CHUNK
            pltpu.sync_copy(acc_sh.at[pl.ds(off, CHUNK)],
                            out_hbm.at[c, pl.ds(off, CHUNK)])

    return k(x2, src3, dst3)


_HIGH = jax.lax.Precision.DEFAULT


def _tc_layer1(epsv, x, aggr, W1, b1):
    blk = 2000

    def body(eps_ref, x_ref, aL_ref, aR_ref, w_ref, b_ref, y_ref, s_ref, q_ref):
        i = pl.program_id(0)
        e = eps_ref[0, 0]
        h = (1.0 + e) * x_ref[...] + jnp.concatenate(
            [aL_ref[0], aR_ref[0]], axis=1)
        y = jnp.dot(h, w_ref[...], preferred_element_type=jnp.float32,
                    precision=_HIGH) + b_ref[...]
        y_ref[...] = y.astype(jnp.bfloat16)

        @pl.when(i == 0)
        def _():
            s_ref[...] = jnp.zeros_like(s_ref)
            q_ref[...] = jnp.zeros_like(q_ref)

        s_ref[...] += jnp.sum(y, axis=0, keepdims=True)
        q_ref[...] += jnp.sum(y * y, axis=0, keepdims=True)

    return pl.pallas_call(
        body,
        grid=(N_NODES // blk,),
        in_specs=[
            pl.BlockSpec((1, 1), lambda i: (0, 0), memory_space=pltpu.SMEM),
            pl.BlockSpec((blk, D_IN), lambda i: (i, 0)),
            pl.BlockSpec((1, blk, HALF), lambda i: (0, i, 0)),
            pl.BlockSpec((1, blk, HALF), lambda i: (1, i, 0)),
            pl.BlockSpec((D_IN, D_HID), lambda i: (0, 0)),
            pl.BlockSpec((1, D_HID), lambda i: (0, 0)),
        ],
        out_specs=[
            pl.BlockSpec((blk, D_HID), lambda i: (i, 0)),
            pl.BlockSpec((1, D_HID), lambda i: (0, 0)),
            pl.BlockSpec((1, D_HID), lambda i: (0, 0)),
        ],
        out_shape=[
            jax.ShapeDtypeStruct((N_NODES, D_HID), jnp.bfloat16),
            jax.ShapeDtypeStruct((1, D_HID), jnp.float32),
            jax.ShapeDtypeStruct((1, D_HID), jnp.float32),
        ],
    )(epsv, x, aggr, aggr, W1, b1)


def _bn_affine(s, q, g, be):
    m = s * (1.0 / N_NODES)
    v = q * (1.0 / N_NODES) - m * m
    a = g * jax.lax.rsqrt(v + 1e-5)
    return a, be - m * a


def _tc_layer2(y1, s1, q1, g1, be1, W2, b2):
    blk = 2000

    def body(y_ref, s1_ref, q1_ref, g_ref, be_ref, w_ref, b_ref,
             y2_ref, s_ref, q_ref):
        i = pl.program_id(0)
        a, b0 = _bn_affine(s1_ref[...], q1_ref[...], g_ref[...], be_ref[...])
        z = jnp.maximum(y_ref[...].astype(jnp.float32) * a + b0, 0.0)
        y2 = jnp.dot(z, w_ref[...], preferred_element_type=jnp.float32,
                     precision=_HIGH) + b_ref[...]
        y2_ref[...] = y2.astype(jnp.bfloat16)

        @pl.when(i == 0)
        def _():
            s_ref[...] = jnp.zeros_like(s_ref)
            q_ref[...] = jnp.zeros_like(q_ref)

        s_ref[...] += jnp.sum(y2, axis=0, keepdims=True)
        q_ref[...] += jnp.sum(y2 * y2, axis=0, keepdims=True)

    vec = pl.BlockSpec((1, D_HID), lambda i: (0, 0))
    return pl.pallas_call(
        body,
        grid=(N_NODES // blk,),
        in_specs=[
            pl.BlockSpec((blk, D_HID), lambda i: (i, 0)),
            vec, vec, vec, vec,
            pl.BlockSpec((D_HID, D_HID), lambda i: (0, 0)),
            vec,
        ],
        out_specs=[
            pl.BlockSpec((blk, D_HID), lambda i: (i, 0)),
            vec, vec,
        ],
        out_shape=[
            jax.ShapeDtypeStruct((N_NODES, D_HID), jnp.bfloat16),
            jax.ShapeDtypeStruct((1, D_HID), jnp.float32),
            jax.ShapeDtypeStruct((1, D_HID), jnp.float32),
        ],
    )(y1, s1, q1, g1, be1, W2, b2)


def _tc_layer3(y2, s2, q2, g2, be2):
    blk = 2000

    def body(y_ref, s2_ref, q2_ref, g_ref, be_ref, o_ref):
        a, b0 = _bn_affine(s2_ref[...], q2_ref[...], g_ref[...], be_ref[...])
        o_ref[...] = jnp.maximum(y_ref[...].astype(jnp.float32) * a + b0, 0.0)

    vec = pl.BlockSpec((1, D_HID), lambda i: (0, 0))
    return pl.pallas_call(
        body,
        grid=(N_NODES // blk,),
        in_specs=[
            pl.BlockSpec((blk, D_HID), lambda i: (i, 0)),
            vec, vec, vec, vec,
        ],
        out_specs=pl.BlockSpec((blk, D_HID), lambda i: (i, 0)),
        out_shape=jax.ShapeDtypeStruct((N_NODES, D_HID), jnp.float32),
    )(y2, s2, q2, g2, be2)


def kernel(x, edge_index, eps, W1, b1, g1, be1, W2, b2, g2, be2):
    E = edge_index.shape[1]
    src = edge_index[0]
    dst = edge_index[1]

    pad = E_PAD - E
    src_p = jnp.concatenate([2 * src, jnp.zeros((pad,), jnp.int32)])
    dst_p = jnp.concatenate([dst, jnp.full((pad,), N_NODES, jnp.int32)])
    src3 = jnp.stack([src_p, src_p + 1]).reshape(
        N_SC, N_TILES, N_CHUNKS, CHUNK)
    dst3 = dst_p.reshape(N_TILES, N_CHUNKS, CHUNK)
    x2 = x.reshape(2 * N_NODES, HALF)

    aggr = _sc_aggregate(x2, src3, dst3)

    epsv = jnp.reshape(eps, (1, 1))
    y1, s1, q1 = _tc_layer1(epsv, x, aggr, W1, jnp.reshape(b1, (1, D_HID)))
    g1v = jnp.reshape(g1, (1, D_HID))
    be1v = jnp.reshape(be1, (1, D_HID))
    y2, s2, q2 = _tc_layer2(y1, s1, q1, g1v, be1v, W2,
                            jnp.reshape(b2, (1, D_HID)))
    g2v = jnp.reshape(g2, (1, D_HID))
    be2v = jnp.reshape(be2, (1, D_HID))
    return _tc_layer3(y2, s2, q2, g2v, be2v)


# speedup vs baseline: 1.1917x; 1.0108x over previous
import functools

import jax
import jax.numpy as jnp
from jax import lax
from jax.experimental import pallas as pl
from jax.experimental.pallas import tpu as pltpu
from jax.experimental.pallas import tpu_sc as plsc

N_NODES = 10000
D_IN = 256
D_HID = 512
N_SC = 2
N_TILES = 16
CHUNK = 128
N_CHUNKS = 79
EDGES_PER_TILE = CHUNK * N_CHUNKS
E_PAD = EDGES_PER_TILE * N_TILES
ACC_ROWS = 10240
ROWS_PER_TILE = ACC_ROWS // N_TILES
HALF = D_IN // 2


def _sc_aggregate(x2, src3, dst3):
    @functools.partial(
        pl.kernel,
        mesh=plsc.VectorSubcoreMesh(core_axis_name="c", subcore_axis_name="s"),
        out_type=jax.ShapeDtypeStruct((N_SC, ACC_ROWS, HALF), jnp.float32),
        scratch_types=[
            pltpu.VMEM((40, CHUNK), jnp.int32),
            pltpu.VMEM((40, CHUNK), jnp.int32),
            pltpu.VMEM((CHUNK, HALF), jnp.float32),
            pltpu.VMEM((CHUNK, HALF), jnp.float32),
            pltpu.VMEM_SHARED((ACC_ROWS, HALF), jnp.float32),
            pltpu.SemaphoreType.DMA,
            pltpu.SemaphoreType.DMA,
        ],
    )
    def k(x2_hbm, src_hbm, dst_hbm, out_hbm, src_v, dst_v, rows0, rows1,
          acc_sh, sem0, sem1):
        c = lax.axis_index("c")
        s = lax.axis_index("s")

        def _zrow(i, _):
            def _zlane(l, _):
                rows0[i, pl.ds(l * 16, 16)] = jnp.zeros((16,), jnp.float32)
                return 0
            return lax.fori_loop(0, HALF // 16, _zlane, 0)

        lax.fori_loop(0, CHUNK, _zrow, 0)
        for kk in range(ROWS_PER_TILE // CHUNK):
            pltpu.sync_copy(
                rows0, acc_sh.at[pl.ds(s * ROWS_PER_TILE + kk * CHUNK, CHUNK)])
        plsc.subcore_barrier()

        for ph, nch in ((0, 40), (1, 39)):
            pltpu.sync_copy(src_hbm.at[c, s, pl.ds(ph * 40, nch)],
                            src_v.at[pl.ds(0, nch)])
            pltpu.sync_copy(dst_hbm.at[s, pl.ds(ph * 40, nch)],
                            dst_v.at[pl.ds(0, nch)])
            pltpu.async_copy(x2_hbm.at[src_v.at[0]], rows0, sem0)
            npairs = nch // 2

            def _step(jj, _):
                j0 = 2 * jj
                j1 = j0 + 1
                pltpu.async_copy(x2_hbm.at[src_v.at[j1]], rows1, sem1)
                pltpu.make_async_copy(
                    x2_hbm.at[src_v.at[j0]], rows0, sem0).wait()
                pltpu.sync_copy(rows0, acc_sh.at[dst_v.at[j0]], add=True)

                @pl.when(j0 + 2 < nch)
                def _():
                    pltpu.async_copy(x2_hbm.at[src_v.at[j0 + 2]], rows0, sem0)

                pltpu.make_async_copy(
                    x2_hbm.at[src_v.at[j1]], rows1, sem1).wait()
                pltpu.sync_copy(rows1, acc_sh.at[dst_v.at[j1]], add=True)
                return 0

            lax.fori_loop(0, npairs, _step, 0)
            if nch % 2 == 1:
                j_last = nch - 1
                pltpu.make_async_copy(
                    x2_hbm.at[src_v.at[j_last]], rows0, sem0).wait()
                pltpu.sync_copy(rows0, acc_sh.at[dst_v.at[j_last]], add=True)
        plsc.subcore_barrier()
        for kk in range(ROWS_PER_TILE // CHUNK):
            off = s * ROWS_PER_TILE + kk * CHUNK
            pltpu.sync_copy(acc_sh.at[pl.ds(off, CHUNK)],
                            out_hbm.at[c, pl.ds(off, CHUNK)])

    return k(x2, src3, dst3)


_HIGH = jax.lax.Precision.DEFAULT


def _tc_layer1(epsv, x, aggr, W1, b1):
    blk = 5000

    def body(eps_ref, x_ref, aL_ref, aR_ref, w_ref, b_ref, y_ref, s_ref, q_ref):
        i = pl.program_id(0)
        e = eps_ref[0, 0]
        h = (1.0 + e) * x_ref[...] + jnp.concatenate(
            [aL_ref[0], aR_ref[0]], axis=1)
        y = jnp.dot(h, w_ref[...], preferred_element_type=jnp.float32,
                    precision=_HIGH) + b_ref[...]
        y_ref[...] = y.astype(jnp.bfloat16)

        @pl.when(i == 0)
        def _():
            s_ref[...] = jnp.zeros_like(s_ref)
            q_ref[...] = jnp.zeros_like(q_ref)

        s_ref[...] += jnp.sum(y, axis=0, keepdims=True)
        q_ref[...] += jnp.sum(y * y, axis=0, keepdims=True)

    return pl.pallas_call(
        body,
        grid=(N_NODES // blk,),
        in_specs=[
            pl.BlockSpec((1, 1), lambda i: (0, 0), memory_space=pltpu.SMEM),
            pl.BlockSpec((blk, D_IN), lambda i: (i, 0)),
            pl.BlockSpec((1, blk, HALF), lambda i: (0, i, 0)),
            pl.BlockSpec((1, blk, HALF), lambda i: (1, i, 0)),
            pl.BlockSpec((D_IN, D_HID), lambda i: (0, 0)),
            pl.BlockSpec((1, D_HID), lambda i: (0, 0)),
        ],
        out_specs=[
            pl.BlockSpec((blk, D_HID), lambda i: (i, 0)),
            pl.BlockSpec((1, D_HID), lambda i: (0, 0)),
            pl.BlockSpec((1, D_HID), lambda i: (0, 0)),
        ],
        out_shape=[
            jax.ShapeDtypeStruct((N_NODES, D_HID), jnp.bfloat16),
            jax.ShapeDtypeStruct((1, D_HID), jnp.float32),
            jax.ShapeDtypeStruct((1, D_HID), jnp.float32),
        ],
    )(epsv, x, aggr, aggr, W1, b1)


def _bn_affine(s, q, g, be):
    m = s * (1.0 / N_NODES)
    v = q * (1.0 / N_NODES) - m * m
    a = g * jax.lax.rsqrt(v + 1e-5)
    return a, be - m * a


def _tc_layer2(y1, s1, q1, g1, be1, W2, b2):
    blk = 5000

    def body(y_ref, s1_ref, q1_ref, g_ref, be_ref, w_ref, b_ref,
             y2_ref, s_ref, q_ref):
        i = pl.program_id(0)
        a, b0 = _bn_affine(s1_ref[...], q1_ref[...], g_ref[...], be_ref[...])
        z = jnp.maximum(y_ref[...].astype(jnp.float32) * a + b0, 0.0)
        y2 = jnp.dot(z, w_ref[...], preferred_element_type=jnp.float32,
                     precision=_HIGH) + b_ref[...]
        y2_ref[...] = y2.astype(jnp.bfloat16)

        @pl.when(i == 0)
        def _():
            s_ref[...] = jnp.zeros_like(s_ref)
            q_ref[...] = jnp.zeros_like(q_ref)

        s_ref[...] += jnp.sum(y2, axis=0, keepdims=True)
        q_ref[...] += jnp.sum(y2 * y2, axis=0, keepdims=True)

    vec = pl.BlockSpec((1, D_HID), lambda i: (0, 0))
    return pl.pallas_call(
        body,
        grid=(N_NODES // blk,),
        in_specs=[
            pl.BlockSpec((blk, D_HID), lambda i: (i, 0)),
            vec, vec, vec, vec,
            pl.BlockSpec((D_HID, D_HID), lambda i: (0, 0)),
            vec,
        ],
        out_specs=[
            pl.BlockSpec((blk, D_HID), lambda i: (i, 0)),
            vec, vec,
        ],
        out_shape=[
            jax.ShapeDtypeStruct((N_NODES, D_HID), jnp.bfloat16),
            jax.ShapeDtypeStruct((1, D_HID), jnp.float32),
            jax.ShapeDtypeStruct((1, D_HID), jnp.float32),
        ],
    )(y1, s1, q1, g1, be1, W2, b2)


def _tc_layer3(y2, s2, q2, g2, be2):
    blk = 5000

    def body(y_ref, s2_ref, q2_ref, g_ref, be_ref, o_ref):
        a, b0 = _bn_affine(s2_ref[...], q2_ref[...], g_ref[...], be_ref[...])
        o_ref[...] = jnp.maximum(y_ref[...].astype(jnp.float32) * a + b0, 0.0)

    vec = pl.BlockSpec((1, D_HID), lambda i: (0, 0))
    return pl.pallas_call(
        body,
        grid=(N_NODES // blk,),
        in_specs=[
            pl.BlockSpec((blk, D_HID), lambda i: (i, 0)),
            vec, vec, vec, vec,
        ],
        out_specs=pl.BlockSpec((blk, D_HID), lambda i: (i, 0)),
        out_shape=jax.ShapeDtypeStruct((N_NODES, D_HID), jnp.float32),
    )(y2, s2, q2, g2, be2)


def kernel(x, edge_index, eps, W1, b1, g1, be1, W2, b2, g2, be2):
    E = edge_index.shape[1]
    src = edge_index[0]
    dst = edge_index[1]

    pad = E_PAD - E
    src_p = jnp.concatenate([2 * src, jnp.zeros((pad,), jnp.int32)])
    dst_p = jnp.concatenate([dst, jnp.full((pad,), N_NODES, jnp.int32)])
    src3 = jnp.stack([src_p, src_p + 1]).reshape(
        N_SC, N_TILES, N_CHUNKS, CHUNK)
    dst3 = dst_p.reshape(N_TILES, N_CHUNKS, CHUNK)
    x2 = x.reshape(2 * N_NODES, HALF)

    aggr = _sc_aggregate(x2, src3, dst3)

    epsv = jnp.reshape(eps, (1, 1))
    y1, s1, q1 = _tc_layer1(epsv, x, aggr, W1, jnp.reshape(b1, (1, D_HID)))
    g1v = jnp.reshape(g1, (1, D_HID))
    be1v = jnp.reshape(be1, (1, D_HID))
    y2, s2, q2 = _tc_layer2(y1, s1, q1, g1v, be1v, W2,
                            jnp.reshape(b2, (1, D_HID)))
    g2v = jnp.reshape(g2, (1, D_HID))
    be2v = jnp.reshape(be2, (1, D_HID))
    return _tc_layer3(y2, s2, q2, g2v, be2v)


# speedup vs baseline: 1.1993x; 1.0064x over previous
import functools

import jax
import jax.numpy as jnp
from jax import lax
from jax.experimental import pallas as pl
from jax.experimental.pallas import tpu as pltpu
from jax.experimental.pallas import tpu_sc as plsc

N_NODES = 10000
D_IN = 256
D_HID = 512
N_SC = 2
N_TILES = 16
CHUNK = 128
N_CHUNKS = 79
EDGES_PER_TILE = CHUNK * N_CHUNKS
E_PAD = EDGES_PER_TILE * N_TILES
ACC_ROWS = 10240
ROWS_PER_TILE = ACC_ROWS // N_TILES
HALF = D_IN // 2


def _sc_aggregate(x2, src3, dst3):
    @functools.partial(
        pl.kernel,
        mesh=plsc.VectorSubcoreMesh(core_axis_name="c", subcore_axis_name="s"),
        out_type=jax.ShapeDtypeStruct((N_SC, ACC_ROWS, HALF), jnp.float32),
        scratch_types=[
            pltpu.VMEM((40, CHUNK), jnp.int32),
            pltpu.VMEM((40, CHUNK), jnp.int32),
            pltpu.VMEM((CHUNK, HALF), jnp.float32),
            pltpu.VMEM((CHUNK, HALF), jnp.float32),
            pltpu.VMEM_SHARED((ACC_ROWS, HALF), jnp.float32),
            pltpu.SemaphoreType.DMA,
            pltpu.SemaphoreType.DMA,
        ],
    )
    def k(x2_hbm, src_hbm, dst_hbm, out_hbm, src_v, dst_v, rows0, rows1,
          acc_sh, sem0, sem1):
        c = lax.axis_index("c")
        s = lax.axis_index("s")

        pltpu.async_copy(src_hbm.at[c, s, pl.ds(0, 40)], src_v, sem0)
        pltpu.async_copy(dst_hbm.at[s, pl.ds(0, 40)], dst_v, sem1)

        def _zrow(i, _):
            def _zlane(l, _):
                rows0[i, pl.ds(l * 16, 16)] = jnp.zeros((16,), jnp.float32)
                return 0
            return lax.fori_loop(0, HALF // 16, _zlane, 0)

        lax.fori_loop(0, CHUNK, _zrow, 0)
        for kk in range(ROWS_PER_TILE // CHUNK):
            pltpu.sync_copy(
                rows0, acc_sh.at[pl.ds(s * ROWS_PER_TILE + kk * CHUNK, CHUNK)])
        pltpu.make_async_copy(src_hbm.at[c, s, pl.ds(0, 40)], src_v, sem0).wait()
        pltpu.make_async_copy(dst_hbm.at[s, pl.ds(0, 40)], dst_v, sem1).wait()
        plsc.subcore_barrier()

        for ph, nch in ((0, 40), (1, 39)):
            if ph > 0:
                pltpu.sync_copy(src_hbm.at[c, s, pl.ds(ph * 40, nch)],
                                src_v.at[pl.ds(0, nch)])
                pltpu.sync_copy(dst_hbm.at[s, pl.ds(ph * 40, nch)],
                                dst_v.at[pl.ds(0, nch)])
            pltpu.async_copy(x2_hbm.at[src_v.at[0]], rows0, sem0)
            npairs = nch // 2

            def _step(jj, _):
                j0 = 2 * jj
                j1 = j0 + 1
                pltpu.async_copy(x2_hbm.at[src_v.at[j1]], rows1, sem1)
                pltpu.make_async_copy(
                    x2_hbm.at[src_v.at[j0]], rows0, sem0).wait()
                pltpu.sync_copy(rows0, acc_sh.at[dst_v.at[j0]], add=True)

                @pl.when(j0 + 2 < nch)
                def _():
                    pltpu.async_copy(x2_hbm.at[src_v.at[j0 + 2]], rows0, sem0)

                pltpu.make_async_copy(
                    x2_hbm.at[src_v.at[j1]], rows1, sem1).wait()
                pltpu.sync_copy(rows1, acc_sh.at[dst_v.at[j1]], add=True)
                return 0

            lax.fori_loop(0, npairs, _step, 0)
            if nch % 2 == 1:
                j_last = nch - 1
                pltpu.make_async_copy(
                    x2_hbm.at[src_v.at[j_last]], rows0, sem0).wait()
                pltpu.sync_copy(rows0, acc_sh.at[dst_v.at[j_last]], add=True)
        plsc.subcore_barrier()
        for kk in range(ROWS_PER_TILE // CHUNK):
            off = s * ROWS_PER_TILE + kk * CHUNK
            pltpu.sync_copy(acc_sh.at[pl.ds(off, CHUNK)],
                            out_hbm.at[c, pl.ds(off, CHUNK)])

    return k(x2, src3, dst3)


_HIGH = jax.lax.Precision.DEFAULT


def _tc_layer1(epsv, x, aggr, W1, b1):
    blk = 5000

    def body(eps_ref, x_ref, aL_ref, aR_ref, w_ref, b_ref, y_ref, s_ref, q_ref):
        i = pl.program_id(0)
        e = eps_ref[0, 0]
        h = (1.0 + e) * x_ref[...] + jnp.concatenate(
            [aL_ref[0], aR_ref[0]], axis=1)
        y = jnp.dot(h, w_ref[...], preferred_element_type=jnp.float32,
                    precision=_HIGH) + b_ref[...]
        y_ref[...] = y.astype(jnp.bfloat16)

        @pl.when(i == 0)
        def _():
            s_ref[...] = jnp.zeros_like(s_ref)
            q_ref[...] = jnp.zeros_like(q_ref)

        s_ref[...] += jnp.sum(y, axis=0, keepdims=True)
        q_ref[...] += jnp.sum(y * y, axis=0, keepdims=True)

    return pl.pallas_call(
        body,
        grid=(N_NODES // blk,),
        in_specs=[
            pl.BlockSpec((1, 1), lambda i: (0, 0), memory_space=pltpu.SMEM),
            pl.BlockSpec((blk, D_IN), lambda i: (i, 0)),
            pl.BlockSpec((1, blk, HALF), lambda i: (0, i, 0)),
            pl.BlockSpec((1, blk, HALF), lambda i: (1, i, 0)),
            pl.BlockSpec((D_IN, D_HID), lambda i: (0, 0)),
            pl.BlockSpec((1, D_HID), lambda i: (0, 0)),
        ],
        out_specs=[
            pl.BlockSpec((blk, D_HID), lambda i: (i, 0)),
            pl.BlockSpec((1, D_HID), lambda i: (0, 0)),
            pl.BlockSpec((1, D_HID), lambda i: (0, 0)),
        ],
        out_shape=[
            jax.ShapeDtypeStruct((N_NODES, D_HID), jnp.bfloat16),
            jax.ShapeDtypeStruct((1, D_HID), jnp.float32),
            jax.ShapeDtypeStruct((1, D_HID), jnp.float32),
        ],
    )(epsv, x, aggr, aggr, W1, b1)


def _bn_affine(s, q, g, be):
    m = s * (1.0 / N_NODES)
    v = q * (1.0 / N_NODES) - m * m
    a = g * jax.lax.rsqrt(v + 1e-5)
    return a, be - m * a


def _tc_layer2(y1, s1, q1, g1, be1, W2, b2):
    blk = 5000

    def body(y_ref, s1_ref, q1_ref, g_ref, be_ref, w_ref, b_ref,
             y2_ref, s_ref, q_ref):
        i = pl.program_id(0)
        a, b0 = _bn_affine(s1_ref[...], q1_ref[...], g_ref[...], be_ref[...])
        z = jnp.maximum(y_ref[...].astype(jnp.float32) * a + b0, 0.0)
        y2 = jnp.dot(z, w_ref[...], preferred_element_type=jnp.float32,
                     precision=_HIGH) + b_ref[...]
        y2_ref[...] = y2.astype(jnp.bfloat16)

        @pl.when(i == 0)
        def _():
            s_ref[...] = jnp.zeros_like(s_ref)
            q_ref[...] = jnp.zeros_like(q_ref)

        s_ref[...] += jnp.sum(y2, axis=0, keepdims=True)
        q_ref[...] += jnp.sum(y2 * y2, axis=0, keepdims=True)

    vec = pl.BlockSpec((1, D_HID), lambda i: (0, 0))
    return pl.pallas_call(
        body,
        grid=(N_NODES // blk,),
        in_specs=[
            pl.BlockSpec((blk, D_HID), lambda i: (i, 0)),
            vec, vec, vec, vec,
            pl.BlockSpec((D_HID, D_HID), lambda i: (0, 0)),
            vec,
        ],
        out_specs=[
            pl.BlockSpec((blk, D_HID), lambda i: (i, 0)),
            vec, vec,
        ],
        out_shape=[
            jax.ShapeDtypeStruct((N_NODES, D_HID), jnp.bfloat16),
            jax.ShapeDtypeStruct((1, D_HID), jnp.float32),
            jax.ShapeDtypeStruct((1, D_HID), jnp.float32),
        ],
    )(y1, s1, q1, g1, be1, W2, b2)


def _tc_layer3(y2, s2, q2, g2, be2):
    blk = 5000

    def body(y_ref, s2_ref, q2_ref, g_ref, be_ref, o_ref):
        a, b0 = _bn_affine(s2_ref[...], q2_ref[...], g_ref[...], be_ref[...])
        o_ref[...] = jnp.maximum(y_ref[...].astype(jnp.float32) * a + b0, 0.0)

    vec = pl.BlockSpec((1, D_HID), lambda i: (0, 0))
    return pl.pallas_call(
        body,
        grid=(N_NODES // blk,),
        in_specs=[
            pl.BlockSpec((blk, D_HID), lambda i: (i, 0)),
            vec, vec, vec, vec,
        ],
        out_specs=pl.BlockSpec((blk, D_HID), lambda i: (i, 0)),
        out_shape=jax.ShapeDtypeStruct((N_NODES, D_HID), jnp.float32),
    )(y2, s2, q2, g2, be2)


def kernel(x, edge_index, eps, W1, b1, g1, be1, W2, b2, g2, be2):
    E = edge_index.shape[1]
    src = edge_index[0]
    dst = edge_index[1]

    pad = E_PAD - E
    src_p = jnp.concatenate([2 * src, jnp.zeros((pad,), jnp.int32)])
    dst_p = jnp.concatenate([dst, jnp.full((pad,), N_NODES, jnp.int32)])
    src3 = jnp.stack([src_p, src_p + 1]).reshape(
        N_SC, N_TILES, N_CHUNKS, CHUNK)
    dst3 = dst_p.reshape(N_TILES, N_CHUNKS, CHUNK)
    x2 = x.reshape(2 * N_NODES, HALF)

    aggr = _sc_aggregate(x2, src3, dst3)

    epsv = jnp.reshape(eps, (1, 1))
    y1, s1, q1 = _tc_layer1(epsv, x, aggr, W1, jnp.reshape(b1, (1, D_HID)))
    g1v = jnp.reshape(g1, (1, D_HID))
    be1v = jnp.reshape(be1, (1, D_HID))
    y2, s2, q2 = _tc_layer2(y1, s1, q1, g1v, be1v, W2,
                            jnp.reshape(b2, (1, D_HID)))
    g2v = jnp.reshape(g2, (1, D_HID))
    be2v = jnp.reshape(be2, (1, D_HID))
    return _tc_layer3(y2, s2, q2, g2v, be2v)
